# baseline (speedup 1.0000x reference)
import functools

import jax
import jax.numpy as jnp
from jax import lax
from jax.experimental import pallas as pl
from jax.experimental.pallas import tpu as pltpu
from jax.experimental.pallas import tpu_sc as plsc

PAD_ID = 0
_LANES = 16


def _make_sc_kernel(n_flat, seq, d):
    nw = 32
    per_w = n_flat // nw
    ch = 32
    n_ch = per_w // ch
    n_vec = d // _LANES

    mesh = plsc.VectorSubcoreMesh(core_axis_name="c", subcore_axis_name="s")

    @functools.partial(
        pl.kernel,
        mesh=mesh,
        out_type=jax.ShapeDtypeStruct((n_flat, d), jnp.float32),
        scratch_types=[
            pltpu.VMEM((per_w,), jnp.int32),
            pltpu.VMEM((ch, d), jnp.float32),
            pltpu.VMEM((ch, d), jnp.float32),
            pltpu.VMEM((ch, d), jnp.float32),
            pltpu.VMEM((ch, d), jnp.float32),
            pltpu.VMEM((ch, d), jnp.float32),
            pltpu.SemaphoreType.DMA,
            pltpu.SemaphoreType.DMA,
            pltpu.SemaphoreType.DMA,
            pltpu.SemaphoreType.DMA,
            pltpu.SemaphoreType.DMA,
            pltpu.SemaphoreType.DMA,
            pltpu.SemaphoreType.DMA,
            pltpu.SemaphoreType.DMA,
            pltpu.SemaphoreType.DMA,
            pltpu.SemaphoreType.DMA,
            pltpu.SemaphoreType.DMA,
        ],
    )
    def emb(x_hbm, table_hbm, pe_hbm, out_hbm,
            idx_v, tok0, tok1, tok2, pe0, pe1,
            g0, g1, g2, h0, h1, h2, p0, p1, s0_, s1_, s2_):
        cid = lax.axis_index("c")
        sid = lax.axis_index("s")
        wid = sid * 2 + cid
        base = wid * per_w
        pe_base = base % seq

        toks = [tok0, tok1, tok2]
        pes = [pe0, pe1]
        gsems = [g0, g1, g2]
        hsems = [h0, h1, h2]
        psems = [p0, p1]
        ssems = [s0_, s1_, s2_]

        pltpu.sync_copy(x_hbm.at[pl.ds(base, per_w)], idx_v)

        gd, pd, sd = {}, {}, {}

        hf = ch // 2

        def start_gather(c):
            b = c % 3
            gd[c] = (
                pltpu.async_copy(
                    table_hbm.at[idx_v.at[pl.ds(c * ch, hf)]],
                    toks[b].at[pl.ds(0, hf)], gsems[b],
                ),
                pltpu.async_copy(
                    table_hbm.at[idx_v.at[pl.ds(c * ch + hf, hf)]],
                    toks[b].at[pl.ds(hf, hf)], hsems[b],
                ),
            )

        def start_pe(c):
            pb = c % 2
            pd[c] = pltpu.async_copy(
                pe_hbm.at[pl.ds(pe_base + c * ch, ch)], pes[pb], psems[pb]
            )

        start_gather(0)
        start_pe(0)
        for c in range(n_ch):
            b = c % 3
            r0 = c * ch
            if c + 1 < n_ch:
                start_pe(c + 1)
                if c - 2 in sd:
                    sd[c - 2].wait()
                start_gather(c + 1)
            pd[c].wait()

            ms = []
            for g in range(ch // _LANES):
                iv = idx_v[pl.ds(r0 + g * _LANES, _LANES)]
                mv = jnp.where(iv != PAD_ID, 1.0, 0.0)
                ms.extend(mv[r16] for r16 in range(_LANES))

            tok_v, pe_v = toks[b], pes[c % 2]

            for half, dsc in enumerate(gd[c]):
                dsc.wait()
                rows = range(half * hf, (half + 1) * hf)

                def col_body(j, _, tok_v=tok_v, pe_v=pe_v, ms=ms, rows=rows):
                    o = j * _LANES
                    for row in rows:
                        t = tok_v[row, pl.ds(o, _LANES)]
                        p = pe_v[row, pl.ds(o, _LANES)]
                        tok_v[row, pl.ds(o, _LANES)] = t * ms[row] + p
                    return 0

                lax.fori_loop(0, n_vec, col_body, 0)

            sd[c] = pltpu.async_copy(
                tok_v, out_hbm.at[pl.ds(base + r0, ch)], ssems[b]
            )
        for c in (n_ch - 3, n_ch - 2, n_ch - 1):
            sd[c].wait()

    return emb


@jax.jit
def kernel(x, table, pe):
    b, s = x.shape
    d = table.shape[1]
    xf = x.reshape(b * s).astype(jnp.int32)
    emb = _make_sc_kernel(b * s, s, d)
    out = emb(xf, table, pe[:s])
    return out.reshape(b, s, d)


# speedup vs baseline: 1.0367x; 1.0367x over previous
import functools

import jax
import jax.numpy as jnp
from jax import lax
from jax.experimental import pallas as pl
from jax.experimental.pallas import tpu as pltpu
from jax.experimental.pallas import tpu_sc as plsc

PAD_ID = 0
_LANES = 16


def _make_sc_kernel(n_flat, seq, d):
    nw = 32
    per_w = n_flat // nw
    ch = 32
    n_ch = per_w // ch
    n_vec = d // _LANES

    mesh = plsc.VectorSubcoreMesh(core_axis_name="c", subcore_axis_name="s")

    @functools.partial(
        pl.kernel,
        mesh=mesh,
        out_type=jax.ShapeDtypeStruct((n_flat, d), jnp.float32),
        scratch_types=[
            pltpu.VMEM((per_w,), jnp.int32),
            pltpu.VMEM((ch, d), jnp.float32),
            pltpu.VMEM((ch, d), jnp.float32),
            pltpu.VMEM((ch, d), jnp.float32),
            pltpu.VMEM((ch, d), jnp.float32),
            pltpu.VMEM((ch, d), jnp.float32),
            pltpu.SemaphoreType.DMA,
            pltpu.SemaphoreType.DMA,
            pltpu.SemaphoreType.DMA,
            pltpu.SemaphoreType.DMA,
            pltpu.SemaphoreType.DMA,
            pltpu.SemaphoreType.DMA,
            pltpu.SemaphoreType.DMA,
            pltpu.SemaphoreType.DMA,
        ],
    )
    def emb(x_hbm, table_hbm, pe_hbm, out_hbm,
            idx_v, tok0, tok1, tok2, pe0, pe1,
            g0, g1, g2, p0, p1, s0_, s1_, s2_):
        cid = lax.axis_index("c")
        sid = lax.axis_index("s")
        wid = sid * 2 + cid
        base = wid * per_w
        pe_base = base % seq

        toks = [tok0, tok1, tok2]
        pes = [pe0, pe1]
        gsems = [g0, g1, g2]
        psems = [p0, p1]
        ssems = [s0_, s1_, s2_]

        pltpu.sync_copy(x_hbm.at[pl.ds(base, per_w)], idx_v)

        gd, pd, sd = {}, {}, {}

        def start_gather(c):
            b = c % 3
            gd[c] = pltpu.async_copy(
                table_hbm.at[idx_v.at[pl.ds(c * ch, ch)]], toks[b], gsems[b]
            )

        def start_pe(c):
            pb = c % 2
            pd[c] = pltpu.async_copy(
                pe_hbm.at[pl.ds(pe_base + c * ch, ch)], pes[pb], psems[pb]
            )

        start_gather(0)
        start_pe(0)
        for c in range(n_ch):
            b = c % 3
            r0 = c * ch
            if c + 1 < n_ch:
                start_pe(c + 1)
                if c - 2 in sd:
                    sd[c - 2].wait()
                start_gather(c + 1)
            gd[c].wait()
            pd[c].wait()

            ms = []
            for g in range(ch // _LANES):
                iv = idx_v[pl.ds(r0 + g * _LANES, _LANES)]
                mv = jnp.where(iv != PAD_ID, 1.0, 0.0)
                ms.extend(mv[r16] for r16 in range(_LANES))

            tok_v, pe_v = toks[b], pes[c % 2]

            def col_body(j, _, tok_v=tok_v, pe_v=pe_v, ms=ms):
                o = j * _LANES
                for row in range(ch):
                    t = tok_v[row, pl.ds(o, _LANES)]
                    p = pe_v[row, pl.ds(o, _LANES)]
                    tok_v[row, pl.ds(o, _LANES)] = t * ms[row] + p
                return 0

            lax.fori_loop(0, n_vec, col_body, 0)

            sd[c] = pltpu.async_copy(
                tok_v, out_hbm.at[pl.ds(base + r0, ch)], ssems[b]
            )
        for c in (n_ch - 3, n_ch - 2, n_ch - 1):
            sd[c].wait()

    return emb


@jax.jit
def kernel(x, table, pe):
    b, s = x.shape
    d = table.shape[1]
    xf = x.reshape(b * s).astype(jnp.int32)
    emb = _make_sc_kernel(b * s, s, d)
    out = emb(xf, table, pe[:s])
    return out.reshape(b, s, d)
